# Initial kernel scaffold; baseline (speedup 1.0000x reference)
import jax
import jax.numpy as jnp
from jax.experimental import pallas as pl


def kernel(sentences, embedding_matrix, training):
    raise NotImplementedError("write your pallas kernel here")


# speedup vs baseline: 2.6241x; 2.6241x over previous
import functools

import jax
import jax.numpy as jnp
from jax import lax
from jax.experimental import pallas as pl
from jax.experimental.pallas import tpu as pltpu
from jax.experimental.pallas import tpu_sc as plsc

_WORD_DROPOUT = 0.1
_VOCAB = 100000
_D = 128

_NC = 2
_NS = 16
_NW = _NC * _NS
_L = 16

_B = 4096 * 50
_PER_W = _B // _NW
_CHUNK = 128
_NCHUNK = _PER_W // _CHUNK
_BITS_W = 3200


def _sc_body(table_hbm, idx_hbm, bits_hbm, skeep_hbm, sdrop_hbm, out_hbm,
             idx_v, bits_v, skeep_v, sdrop_v, scales_v, rows_v, sem):
    wid = lax.axis_index("s") * _NC + lax.axis_index("c")
    base = wid * _PER_W

    pltpu.sync_copy(idx_hbm.at[wid], idx_v)
    pltpu.sync_copy(bits_hbm, bits_v)
    pltpu.sync_copy(skeep_hbm, skeep_v)
    pltpu.sync_copy(sdrop_hbm, sdrop_v)

    @pl.loop(0, _NCHUNK)
    def _chunk(c):
        cp = pltpu.async_copy(table_hbm.at[idx_v.at[c]], rows_v, sem)

        s_keep = skeep_v[...]
        s_drop = sdrop_v[...]
        for p in range(_CHUNK // _L):
            iv = idx_v[c, pl.ds(p * _L, _L)]
            w = plsc.load_gather(bits_v, [lax.shift_right_logical(iv, 5)])
            bit = lax.shift_right_logical(w, iv & 31) & 1
            scales_v[pl.ds(p * _L, _L)] = jnp.where(bit == 1, s_keep, s_drop)

        cp.wait()

        @pl.loop(0, _CHUNK, unroll=2)
        def _row(r):
            sc = plsc.load_gather(scales_v, [jnp.full((_L,), r, jnp.int32)])
            for p in range(_D // _L):
                rows_v[r, pl.ds(p * _L, _L)] = rows_v[r, pl.ds(p * _L, _L)] * sc

        pltpu.sync_copy(rows_v, out_hbm.at[pl.ds(base + c * _CHUNK, _CHUNK)])


def kernel(sentences, embedding_matrix, training):
    p = _WORD_DROPOUT
    keep = jax.random.bernoulli(
        jax.random.key(42), 1.0 - p, (embedding_matrix.shape[0], 1))[:, 0]
    keep_pad = jnp.zeros((_BITS_W * 32,), jnp.uint32).at[:_VOCAB].set(
        keep.astype(jnp.uint32))
    bits = (keep_pad.reshape(_BITS_W, 32)
            << jnp.arange(32, dtype=jnp.uint32)[None, :]).sum(
                axis=1, dtype=jnp.uint32).astype(jnp.int32)
    s_drop = jnp.full((_L,), jnp.where(training, 0.0, 1.0), jnp.float32)
    s_keep = jnp.full((_L,), jnp.where(training, 1.0 / (1.0 - p), 1.0),
                      jnp.float32)

    idx = sentences.reshape(_NW, _NCHUNK, _CHUNK)

    mesh = plsc.VectorSubcoreMesh(core_axis_name="c", subcore_axis_name="s")
    out = pl.kernel(
        _sc_body,
        out_type=jax.ShapeDtypeStruct((_B, _D), jnp.float32),
        mesh=mesh,
        compiler_params=pltpu.CompilerParams(needs_layout_passes=False),
        scratch_types=[
            pltpu.VMEM((_NCHUNK, _CHUNK), jnp.int32),
            pltpu.VMEM((_BITS_W,), jnp.int32),
            pltpu.VMEM((_L,), jnp.float32),
            pltpu.VMEM((_L,), jnp.float32),
            pltpu.VMEM((_CHUNK,), jnp.float32),
            pltpu.VMEM((_CHUNK, _D), jnp.float32),
            pltpu.SemaphoreType.DMA,
        ],
    )(embedding_matrix, idx, bits, s_keep, s_drop)
    return out.reshape(sentences.shape[0], sentences.shape[1], _D)


# speedup vs baseline: 3.3606x; 1.2807x over previous
import functools

import jax
import jax.numpy as jnp
from jax import lax
from jax.experimental import pallas as pl
from jax.experimental.pallas import tpu as pltpu
from jax.experimental.pallas import tpu_sc as plsc

_WORD_DROPOUT = 0.1
_VOCAB = 100000
_D = 128

_NC = 2
_NS = 16
_NW = _NC * _NS
_L = 16

_B = 4096 * 50
_PER_W = _B // _NW
_CHUNK = 64
_NCHUNK = _PER_W // _CHUNK
_NBUF = 4
_BITS_W = 3200


def _sc_body(table_hbm, idx_hbm, bits_hbm, skeep_hbm, sdrop_hbm, out_hbm,
             idx_v, bits_v, skeep_v, sdrop_v, scales_v, rows_v,
             gsems, ssems):
    wid = lax.axis_index("s") * _NC + lax.axis_index("c")
    base = wid * _PER_W

    pltpu.sync_copy(idx_hbm.at[wid], idx_v)
    pltpu.sync_copy(bits_hbm, bits_v)
    pltpu.sync_copy(skeep_hbm, skeep_v)
    pltpu.sync_copy(sdrop_hbm, sdrop_v)

    def start_gather(c, slot):
        pltpu.async_copy(table_hbm.at[idx_v.at[c]], rows_v.at[slot],
                         gsems.at[slot])

    def wait_gather(c, slot):
        pltpu.make_async_copy(table_hbm.at[idx_v.at[c]], rows_v.at[slot],
                              gsems.at[slot]).wait()

    def out_slice(c):
        return out_hbm.at[pl.ds(base + c * _CHUNK, _CHUNK)]

    def start_scatter(c, slot):
        pltpu.async_copy(rows_v.at[slot], out_slice(c), ssems.at[slot])

    def drain_scatter(c, slot):
        pltpu.make_async_copy(rows_v.at[slot], out_slice(c),
                              ssems.at[slot]).wait()

    start_gather(0, 0)
    start_gather(1, 1)

    @pl.loop(0, _NCHUNK // _NBUF)
    def _quad(q):
        cb = q * _NBUF
        for j in range(_NBUF):
            c = cb + j
            nslot = (j + 2) % _NBUF

            @pl.when(c >= 2)
            def _():
                drain_scatter(c - 2, nslot)

            @pl.when(c + 2 < _NCHUNK)
            def _():
                start_gather(c + 2, nslot)

            s_keep = skeep_v[...]
            s_drop = sdrop_v[...]
            for p in range(_CHUNK // _L):
                iv = idx_v[c, pl.ds(p * _L, _L)]
                w = plsc.load_gather(bits_v, [lax.shift_right_logical(iv, 5)])
                bit = lax.shift_right_logical(w, iv & 31) & 1
                scales_v[pl.ds(p * _L, _L)] = jnp.where(bit == 1, s_keep,
                                                        s_drop)

            wait_gather(c, j)

            @pl.loop(0, _CHUNK, unroll=2)
            def _row(r):
                sc = plsc.load_gather(scales_v,
                                      [jnp.full((_L,), r, jnp.int32)])
                for p in range(_D // _L):
                    rows_v[j, r, pl.ds(p * _L, _L)] = (
                        rows_v[j, r, pl.ds(p * _L, _L)] * sc)

            start_scatter(c, j)

    drain_scatter(_NCHUNK - 2, (_NCHUNK - 2) % _NBUF)
    drain_scatter(_NCHUNK - 1, (_NCHUNK - 1) % _NBUF)


def kernel(sentences, embedding_matrix, training):
    p = _WORD_DROPOUT
    keep = jax.random.bernoulli(
        jax.random.key(42), 1.0 - p, (embedding_matrix.shape[0], 1))[:, 0]
    keep_pad = jnp.zeros((_BITS_W * 32,), jnp.uint32).at[:_VOCAB].set(
        keep.astype(jnp.uint32))
    bits = (keep_pad.reshape(_BITS_W, 32)
            << jnp.arange(32, dtype=jnp.uint32)[None, :]).sum(
                axis=1, dtype=jnp.uint32).astype(jnp.int32)
    s_drop = jnp.full((_L,), jnp.where(training, 0.0, 1.0), jnp.float32)
    s_keep = jnp.full((_L,), jnp.where(training, 1.0 / (1.0 - p), 1.0),
                      jnp.float32)

    idx = sentences.reshape(_NW, _NCHUNK, _CHUNK)

    mesh = plsc.VectorSubcoreMesh(core_axis_name="c", subcore_axis_name="s")
    out = pl.kernel(
        _sc_body,
        out_type=jax.ShapeDtypeStruct((_B, _D), jnp.float32),
        mesh=mesh,
        compiler_params=pltpu.CompilerParams(needs_layout_passes=False),
        scratch_types=[
            pltpu.VMEM((_NCHUNK, _CHUNK), jnp.int32),
            pltpu.VMEM((_BITS_W,), jnp.int32),
            pltpu.VMEM((_L,), jnp.float32),
            pltpu.VMEM((_L,), jnp.float32),
            pltpu.VMEM((_CHUNK,), jnp.float32),
            pltpu.VMEM((_NBUF, _CHUNK, _D), jnp.float32),
            pltpu.SemaphoreType.DMA((_NBUF,)),
            pltpu.SemaphoreType.DMA((_NBUF,)),
        ],
    )(embedding_matrix, idx, bits, s_keep, s_drop)
    return out.reshape(sentences.shape[0], sentences.shape[1], _D)
